# Initial kernel scaffold; baseline (speedup 1.0000x reference)
import jax
import jax.numpy as jnp
from jax.experimental import pallas as pl


def kernel(hidden_states, Wr, Wg, Wu, Wd):
    raise NotImplementedError("write your pallas kernel here")


# speedup vs baseline: 1.1668x; 1.1668x over previous
import functools

import jax
import jax.numpy as jnp
from jax.experimental import pallas as pl
from jax.experimental.pallas import tpu as pltpu

S = 2048
D_MODEL = 1024
D_FF = 4096
E = 16
BLK = D_FF // E
FF_TILE = 512
N_FF = D_FF // FF_TILE
EXP_PER_TILE = FF_TILE // BLK


def _ffn_kernel(h_ref, wr_ref, wg_ref, wu_ref, wd_ref, out_ref, routing_ref):
    f = pl.program_id(0)

    @pl.when(f == 0)
    def _router():
        logits = jax.lax.dot_general(
            h_ref[...], wr_ref[...],
            dimension_numbers=(((1,), (1,)), ((), ())),
            preferred_element_type=jnp.float32,
        )
        r = jnp.maximum(logits, 0.0)
        r = r / (jnp.sum(r, axis=1, keepdims=True) + 1e-6)
        routing_ref[...] = r.astype(jnp.bfloat16)

    h = h_ref[...]
    gate = jax.lax.dot_general(
        h, wg_ref[...], dimension_numbers=(((1,), (1,)), ((), ())),
        preferred_element_type=jnp.float32,
    )
    up = jax.lax.dot_general(
        h, wu_ref[...], dimension_numbers=(((1,), (1,)), ((), ())),
        preferred_element_type=jnp.float32,
    )

    col_expert = (
        jax.lax.broadcasted_iota(jnp.int32, (E, FF_TILE), 1) + f * FF_TILE
    ) // BLK
    row_expert = jax.lax.broadcasted_iota(jnp.int32, (E, FF_TILE), 0)
    onehot = (row_expert == col_expert).astype(jnp.bfloat16)
    scale = jax.lax.dot_general(
        routing_ref[...], onehot, dimension_numbers=(((1,), (0,)), ((), ())),
        preferred_element_type=jnp.float32,
    )

    inter = gate * jax.nn.sigmoid(gate) * up * scale
    dp = jax.lax.dot_general(
        inter.astype(jnp.bfloat16), wd_ref[...],
        dimension_numbers=(((1,), (1,)), ((), ())),
        preferred_element_type=jnp.float32,
    )

    @pl.when(f == 0)
    def _init():
        out_ref[...] = dp

    @pl.when(f > 0)
    def _acc():
        out_ref[...] += dp


@functools.partial(jax.jit, static_argnames=("interpret",))
def _run(h2d, wr, wg, wu, wd, interpret=False):
    hb = h2d.astype(jnp.bfloat16)
    out = pl.pallas_call(
        _ffn_kernel,
        grid=(N_FF,),
        in_specs=[
            pl.BlockSpec((S, D_MODEL), lambda f: (0, 0)),
            pl.BlockSpec((E, D_MODEL), lambda f: (0, 0)),
            pl.BlockSpec((FF_TILE, D_MODEL), lambda f: (f, 0)),
            pl.BlockSpec((FF_TILE, D_MODEL), lambda f: (f, 0)),
            pl.BlockSpec((D_MODEL, FF_TILE), lambda f: (0, f)),
        ],
        out_specs=pl.BlockSpec((S, D_MODEL), lambda f: (0, 0)),
        out_shape=jax.ShapeDtypeStruct((S, D_MODEL), jnp.float32),
        scratch_shapes=[pltpu.VMEM((S, E), jnp.bfloat16)],
        interpret=interpret,
    )(
        hb,
        wr.astype(jnp.bfloat16),
        wg.astype(jnp.bfloat16),
        wu.astype(jnp.bfloat16),
        wd.astype(jnp.bfloat16),
    )
    return out


def kernel(hidden_states, Wr, Wg, Wu, Wd):
    b, s, d = hidden_states.shape
    out = _run(hidden_states.reshape(s, d), Wr, Wg, Wu, Wd)
    return out.reshape(b, s, d)
